# baseline (speedup 1.0000x reference)
import functools

import jax
import jax.numpy as jnp
from jax import lax
from jax.experimental import pallas as pl
from jax.experimental.pallas import tpu as pltpu
from jax.experimental.pallas import tpu_sc as plsc

NC = 2
NS = 16
NW = NC * NS
K = 80
RING = 4
LANES = 16


def _sc_message_pass(table, srcs, attrs, dsts, vb, n_nodes, n_acc, ch):
    D = table.shape[1]
    ncol = D // LANES

    def body(table_r, srcs_r, attrs_r, dsts_r, vb_r, out_r,
             rows0, rows1, rows2, rows3, ebuf_i, ebuf_a, ebuf_d, vb_v, acc,
             gs0, gs1, gs2, gs3, ss0, ss1, ss2, ss3,
             es0, es1, es2, es3, ds0, ds1, ds2, ds3):
        c = lax.axis_index("c")
        s = lax.axis_index("s")
        w = s * NC + c
        rows_b = (rows0, rows1, rows2, rows3)
        gs_b = (gs0, gs1, gs2, gs3)
        ss_b = (ss0, ss1, ss2, ss3)
        es_b = (es0, es1, es2, es3)
        ds_b = (ds0, ds1, ds2, ds3)

        def fetch_edges(g, b):
            pltpu.async_copy(srcs_r.at[w, g], ebuf_i.at[b], es_b[b])
            pltpu.async_copy(attrs_r.at[w, g], ebuf_a.at[b], es_b[b])

        def wait_edges(g, b):
            pltpu.make_async_copy(srcs_r.at[w, g], ebuf_i.at[b],
                                  es_b[b]).wait()
            pltpu.make_async_copy(attrs_r.at[w, g], ebuf_a.at[b],
                                  es_b[b]).wait()

        def fetch_dst(g, b):
            pltpu.async_copy(dsts_r.at[w, g], ebuf_d.at[b], ds_b[b])

        def wait_dst(g, b):
            pltpu.make_async_copy(dsts_r.at[w, g], ebuf_d.at[b],
                                  ds_b[b]).wait()

        for b in range(RING):
            fetch_edges(b, b)
        for b in range(2):
            fetch_dst(b, b)
        pltpu.sync_copy(vb_r, vb_v)

        for b in range(2):
            wait_edges(b, b)
            pltpu.async_copy(table_r.at[ebuf_i.at[b]], rows_b[b], gs_b[b])

        zero = jnp.zeros((LANES,), jnp.float32)

        @pl.loop(0, K)
        def _(j):
            for col in range(ncol):
                rows3[j, pl.ds(col * LANES, LANES)] = zero

        zr = n_acc // NS
        zfull = zr // K

        @pl.loop(0, zfull)
        def _(t):
            pltpu.sync_copy(rows3, acc.at[pl.ds(s * zr + t * K, K)])

        zrem = zr - zfull * K
        if zrem:
            pltpu.sync_copy(rows3.at[pl.ds(0, zrem)],
                            acc.at[pl.ds(s * zr + zfull * K, zrem)])

        plsc.subcore_barrier()

        vcols = [vb_v[0, pl.ds(col * LANES, LANES)] for col in range(ncol)]
        bcols = [vb_v[1, pl.ds(col * LANES, LANES)] for col in range(ncol)]

        @pl.loop(0, ch, step=RING)
        def _(g):
            for b in range(RING):
                gg = g + b
                bp2 = (b + 2) % RING
                rows = rows_b[b]

                pltpu.make_async_copy(table_r.at[ebuf_i.at[b]], rows,
                                      gs_b[b]).wait()

                @pl.loop(0, K // LANES)
                def _(jj):
                    a_vec = ebuf_a[b, pl.ds(jj * LANES, LANES)]
                    base = jj * LANES
                    for t in range(LANES):
                        ab = jnp.full((LANES,), a_vec[t], jnp.float32)
                        j = base + t
                        for col in range(ncol):
                            sl = pl.ds(col * LANES, LANES)
                            tt = ab * vcols[col] + bcols[col]
                            rows[j, sl] = jnp.maximum(rows[j, sl] + tt, 0.0)

                wait_dst(gg, b)
                pltpu.async_copy(rows, acc.at[ebuf_d.at[b]], ss_b[b], add=True)

                @pl.when(gg + RING < ch)
                def _():
                    fetch_edges(gg + RING, b)

                @pl.when(gg + 2 < ch)
                def _():
                    @pl.when(gg >= 2)
                    def _():
                        pltpu.make_async_copy(rows_b[bp2],
                                              acc.at[ebuf_d.at[bp2]],
                                              ss_b[bp2]).wait()

                    fetch_dst(gg + 2, bp2)
                    wait_edges(gg + 2, bp2)
                    pltpu.async_copy(table_r.at[ebuf_i.at[bp2]],
                                     rows_b[bp2], gs_b[bp2])

        for gg in range(ch - 4, ch):
            b = gg % RING
            pltpu.make_async_copy(rows_b[b], acc.at[ebuf_d.at[b]],
                                  ss_b[b]).wait()

        plsc.subcore_barrier()

        zr2 = n_acc // NS
        pltpu.sync_copy(acc.at[pl.ds(s * zr2, zr2)],
                        out_r.at[c, pl.ds(s * zr2, zr2)])

    mesh = plsc.VectorSubcoreMesh(core_axis_name="c", subcore_axis_name="s")
    f = pl.kernel(
        body,
        out_type=jax.ShapeDtypeStruct((NC, n_acc, D), jnp.float32),
        mesh=mesh,
        scratch_types=(
            [pltpu.VMEM((K, D), jnp.float32)] * 4
            + [
                pltpu.VMEM((RING, K), jnp.int32),
                pltpu.VMEM((RING, K), jnp.float32),
                pltpu.VMEM((RING, K), jnp.int32),
                pltpu.VMEM((2, D), jnp.float32),
                pltpu.VMEM_SHARED((n_acc, D), jnp.float32),
            ]
            + [pltpu.SemaphoreType.DMA] * 16
        ),
    )
    return f(table, srcs, attrs, dsts, vb)


def _tc_mlp(x, agg, scal, wa, ba, wb, bb):
    n, d = x.shape
    h = wa.shape[1]
    br = 1000

    def body(scal_r, x_r, a_r, wa_r, ba_r, wb_r, bb_r, o_r):
        u = scal_r[0, 0] * x_r[...] + a_r[0] + a_r[1]
        t = jnp.dot(u, wa_r[...], preferred_element_type=jnp.float32) + ba_r[...]
        t = jnp.maximum(t, 0.0)
        o = jnp.dot(t, wb_r[...], preferred_element_type=jnp.float32) + bb_r[...]
        o_r[...] = jnp.maximum(o, 0.0)

    return pl.pallas_call(
        body,
        grid=(n // br,),
        in_specs=[
            pl.BlockSpec(memory_space=pltpu.SMEM),
            pl.BlockSpec((br, d), lambda i: (i, 0)),
            pl.BlockSpec((NC, br, d), lambda i: (0, i, 0)),
            pl.BlockSpec((d, h), lambda i: (0, 0)),
            pl.BlockSpec((1, h), lambda i: (0, 0)),
            pl.BlockSpec((h, h), lambda i: (0, 0)),
            pl.BlockSpec((1, h), lambda i: (0, 0)),
        ],
        out_specs=pl.BlockSpec((br, h), lambda i: (i, 0)),
        out_shape=jax.ShapeDtypeStruct((n, h), jnp.float32),
    )(scal, x, agg, wa, ba.reshape(1, h), wb, bb.reshape(1, h))


def _tc_final(x, agg, scal, wa, ba, wb, bb, lw, lb):
    n, d = x.shape
    h = wa.shape[1]
    c = lw.shape[1]
    br = 1000

    def body(scal_r, x_r, a_r, wa_r, ba_r, wb_r, bb_r, lw_r, lb_r, o_r):
        u = scal_r[0, 0] * x_r[...] + a_r[0] + a_r[1]
        t = jnp.dot(u, wa_r[...], preferred_element_type=jnp.float32) + ba_r[...]
        t = jnp.maximum(t, 0.0)
        m = jnp.dot(t, wb_r[...], preferred_element_type=jnp.float32) + bb_r[...]
        m = jnp.maximum(m, 0.0)
        logits = jnp.dot(m, lw_r[...], preferred_element_type=jnp.float32) + lb_r[...]
        mx = jnp.max(logits, axis=1, keepdims=True)
        l = logits - mx
        lse = jnp.log(jnp.sum(jnp.exp(l), axis=1, keepdims=True))
        o_r[...] = l - lse

    return pl.pallas_call(
        body,
        grid=(n // br,),
        in_specs=[
            pl.BlockSpec(memory_space=pltpu.SMEM),
            pl.BlockSpec((br, d), lambda i: (i, 0)),
            pl.BlockSpec((NC, br, d), lambda i: (0, i, 0)),
            pl.BlockSpec((d, h), lambda i: (0, 0)),
            pl.BlockSpec((1, h), lambda i: (0, 0)),
            pl.BlockSpec((h, h), lambda i: (0, 0)),
            pl.BlockSpec((1, h), lambda i: (0, 0)),
            pl.BlockSpec((h, c), lambda i: (0, 0)),
            pl.BlockSpec((1, c), lambda i: (0, 0)),
        ],
        out_specs=pl.BlockSpec((br, c), lambda i: (i, 0)),
        out_shape=jax.ShapeDtypeStruct((n, c), jnp.float32),
    )(scal, x, agg, wa, ba.reshape(1, h), wb, bb.reshape(1, h),
      lw, lb.reshape(1, c))


def kernel(x, edge_index, edge_attr, w1a, b1a, w1b, b1b, elin1_w, elin1_b, eps1,
           w2a, b2a, w2b, b2b, elin2_w, elin2_b, eps2, lin_w, lin_b):
    n, d = x.shape
    e = edge_index.shape[1]
    ch = -(-e // (NW * K))
    ch += (-ch) % RING
    e_pad = NW * ch * K

    src = edge_index[0]
    dst = edge_index[1]
    a = edge_attr[:, 0]
    pad = e_pad - e
    srcs = jnp.pad(src, (0, pad)).reshape(NW, ch, K)
    dsts = jnp.pad(dst, (0, pad), constant_values=n).reshape(NW, ch, K)
    attrs = jnp.pad(a, (0, pad)).reshape(NW, ch, K)

    n_acc = NS * 8 * (-(-(n + 1) // (NS * 8)))

    vb1 = jnp.concatenate([elin1_w, elin1_b[None, :]], axis=0)
    agg1 = _sc_message_pass(x, srcs, attrs, dsts, vb1, n, n_acc, ch)
    s1 = jnp.reshape(1.0 + eps1, (1, 1))
    h1 = _tc_mlp(x, agg1, s1, w1a, b1a, w1b, b1b)

    vb2 = jnp.concatenate([elin2_w, elin2_b[None, :]], axis=0)
    agg2 = _sc_message_pass(h1, srcs, attrs, dsts, vb2, n, n_acc, ch)
    s2 = jnp.reshape(1.0 + eps2, (1, 1))
    return _tc_final(h1, agg2, s2, w2a, b2a, w2b, b2b, lin_w, lin_b)


# speedup vs baseline: 1.8066x; 1.8066x over previous
import functools

import jax
import jax.numpy as jnp
from jax import lax
from jax.experimental import pallas as pl
from jax.experimental.pallas import tpu as pltpu
from jax.experimental.pallas import tpu_sc as plsc

NC = 2
NS = 16
NW = NC * NS
K = 112
RING = 3
LANES = 16
ASCALE = float(2 ** 24)


def _sc_message_pass(table, sa, dsts, vb, n_nodes, n_acc, ch):
    D = table.shape[1]
    ncol = D // LANES

    def body(table_r, sa_r, dsts_r, vb_r, out_r,
             rows0, rows1, rows2, ebuf_sa, ebuf_d, vb_v, acc,
             gs0, gs1, gs2, ss0, ss1, ss2, es0, es1, es2, ds0, ds1, ds2):
        c = lax.axis_index("c")
        s = lax.axis_index("s")
        w = s * NC + c
        rows_b = (rows0, rows1, rows2)
        gs_b = (gs0, gs1, gs2)
        ss_b = (ss0, ss1, ss2)
        es_b = (es0, es1, es2)
        ds_b = (ds0, ds1, ds2)

        def fetch_edges(g, b):
            pltpu.async_copy(sa_r.at[w, g], ebuf_sa.at[pl.ds(2 * b, 2)],
                             es_b[b])

        def wait_edges(g, b):
            pltpu.make_async_copy(sa_r.at[w, g], ebuf_sa.at[pl.ds(2 * b, 2)],
                                  es_b[b]).wait()

        def fetch_dst(g, b):
            pltpu.async_copy(dsts_r.at[w, g], ebuf_d.at[b], ds_b[b])

        def wait_dst(g, b):
            pltpu.make_async_copy(dsts_r.at[w, g], ebuf_d.at[b],
                                  ds_b[b]).wait()

        for b in range(RING):
            fetch_edges(b, b)
        for b in range(2):
            fetch_dst(b, b)
        pltpu.sync_copy(vb_r, vb_v)

        for b in range(2):
            wait_edges(b, b)
            pltpu.async_copy(table_r.at[ebuf_sa.at[2 * b]], rows_b[b],
                             gs_b[b])

        zero = jnp.zeros((LANES,), jnp.float32)

        @pl.loop(0, K)
        def _(j):
            for col in range(ncol):
                rows2[j, pl.ds(col * LANES, LANES)] = zero

        zr = n_acc // NS
        zfull = zr // K

        @pl.loop(0, zfull)
        def _(t):
            pltpu.sync_copy(rows2, acc.at[pl.ds(s * zr + t * K, K)])

        zrem = zr - zfull * K
        if zrem:
            pltpu.sync_copy(rows2.at[pl.ds(0, zrem)],
                            acc.at[pl.ds(s * zr + zfull * K, zrem)])

        plsc.subcore_barrier()

        vcols = [vb_v[0, pl.ds(col * LANES, LANES)] for col in range(ncol)]
        bcols = [vb_v[1, pl.ds(col * LANES, LANES)] for col in range(ncol)]

        ainv = jnp.float32(1.0 / ASCALE)

        @pl.loop(0, ch, step=RING)
        def _(g):
            for b in range(RING):
                gg = g + b
                bp = (b + RING - 1) % RING
                rows = rows_b[b]

                pltpu.make_async_copy(table_r.at[ebuf_sa.at[2 * b]], rows,
                                      gs_b[b]).wait()

                @pl.loop(0, K // LANES)
                def _(jj):
                    a_fix = ebuf_sa[2 * b + 1, pl.ds(jj * LANES, LANES)]
                    a_vec = a_fix.astype(jnp.float32) * ainv
                    base = jj * LANES
                    for t in range(LANES):
                        ab = jnp.full((LANES,), a_vec[t], jnp.float32)
                        j = base + t
                        for col in range(ncol):
                            sl = pl.ds(col * LANES, LANES)
                            tt = ab * vcols[col] + bcols[col]
                            rows[j, sl] = jnp.maximum(rows[j, sl] + tt, 0.0)

                wait_dst(gg, b)
                pltpu.async_copy(rows, acc.at[ebuf_d.at[b]], ss_b[b], add=True)

                @pl.when(gg + RING < ch)
                def _():
                    fetch_edges(gg + RING, b)

                @pl.when(jnp.logical_and(gg >= 1, gg + 2 < ch))
                def _():
                    pltpu.make_async_copy(rows_b[bp],
                                          acc.at[ebuf_d.at[bp]],
                                          ss_b[bp]).wait()

                @pl.when(gg + 2 < ch)
                def _():
                    fetch_dst(gg + 2, bp)
                    wait_edges(gg + 2, bp)
                    pltpu.async_copy(table_r.at[ebuf_sa.at[2 * bp]],
                                     rows_b[bp], gs_b[bp])

        for gg in range(ch - 3, ch):
            b = gg % RING
            pltpu.make_async_copy(rows_b[b], acc.at[ebuf_d.at[b]],
                                  ss_b[b]).wait()

        plsc.subcore_barrier()

        zr2 = n_acc // NS
        pltpu.sync_copy(acc.at[pl.ds(s * zr2, zr2)],
                        out_r.at[c, pl.ds(s * zr2, zr2)])

    mesh = plsc.VectorSubcoreMesh(core_axis_name="c", subcore_axis_name="s")
    f = pl.kernel(
        body,
        out_type=jax.ShapeDtypeStruct((NC, n_acc, D), jnp.float32),
        mesh=mesh,
        scratch_types=(
            [pltpu.VMEM((K, D), jnp.float32)] * 3
            + [
                pltpu.VMEM((2 * RING, K), jnp.int32),
                pltpu.VMEM((RING, K), jnp.int32),
                pltpu.VMEM((2, D), jnp.float32),
                pltpu.VMEM_SHARED((n_acc, D), jnp.float32),
            ]
            + [pltpu.SemaphoreType.DMA] * 12
        ),
    )
    return f(table, sa, dsts, vb)


def _tc_mlp(x, agg, scal, wa, ba, wb, bb):
    n, d = x.shape
    h = wa.shape[1]
    br = 1000

    def body(scal_r, x_r, a_r, wa_r, ba_r, wb_r, bb_r, o_r):
        u = scal_r[0, 0] * x_r[...] + a_r[0] + a_r[1]
        t = jnp.dot(u, wa_r[...], preferred_element_type=jnp.float32) + ba_r[...]
        t = jnp.maximum(t, 0.0)
        o = jnp.dot(t, wb_r[...], preferred_element_type=jnp.float32) + bb_r[...]
        o_r[...] = jnp.maximum(o, 0.0)

    return pl.pallas_call(
        body,
        grid=(n // br,),
        in_specs=[
            pl.BlockSpec(memory_space=pltpu.SMEM),
            pl.BlockSpec((br, d), lambda i: (i, 0)),
            pl.BlockSpec((NC, br, d), lambda i: (0, i, 0)),
            pl.BlockSpec((d, h), lambda i: (0, 0)),
            pl.BlockSpec((1, h), lambda i: (0, 0)),
            pl.BlockSpec((h, h), lambda i: (0, 0)),
            pl.BlockSpec((1, h), lambda i: (0, 0)),
        ],
        out_specs=pl.BlockSpec((br, h), lambda i: (i, 0)),
        out_shape=jax.ShapeDtypeStruct((n, h), jnp.float32),
    )(scal, x, agg, wa, ba.reshape(1, h), wb, bb.reshape(1, h))


def _tc_final(x, agg, scal, wa, ba, wb, bb, lw, lb):
    n, d = x.shape
    h = wa.shape[1]
    c = lw.shape[1]
    br = 1000

    def body(scal_r, x_r, a_r, wa_r, ba_r, wb_r, bb_r, lw_r, lb_r, o_r):
        u = scal_r[0, 0] * x_r[...] + a_r[0] + a_r[1]
        t = jnp.dot(u, wa_r[...], preferred_element_type=jnp.float32) + ba_r[...]
        t = jnp.maximum(t, 0.0)
        m = jnp.dot(t, wb_r[...], preferred_element_type=jnp.float32) + bb_r[...]
        m = jnp.maximum(m, 0.0)
        logits = jnp.dot(m, lw_r[...], preferred_element_type=jnp.float32) + lb_r[...]
        mx = jnp.max(logits, axis=1, keepdims=True)
        l = logits - mx
        lse = jnp.log(jnp.sum(jnp.exp(l), axis=1, keepdims=True))
        o_r[...] = l - lse

    return pl.pallas_call(
        body,
        grid=(n // br,),
        in_specs=[
            pl.BlockSpec(memory_space=pltpu.SMEM),
            pl.BlockSpec((br, d), lambda i: (i, 0)),
            pl.BlockSpec((NC, br, d), lambda i: (0, i, 0)),
            pl.BlockSpec((d, h), lambda i: (0, 0)),
            pl.BlockSpec((1, h), lambda i: (0, 0)),
            pl.BlockSpec((h, h), lambda i: (0, 0)),
            pl.BlockSpec((1, h), lambda i: (0, 0)),
            pl.BlockSpec((h, c), lambda i: (0, 0)),
            pl.BlockSpec((1, c), lambda i: (0, 0)),
        ],
        out_specs=pl.BlockSpec((br, c), lambda i: (i, 0)),
        out_shape=jax.ShapeDtypeStruct((n, c), jnp.float32),
    )(scal, x, agg, wa, ba.reshape(1, h), wb, bb.reshape(1, h),
      lw, lb.reshape(1, c))


def kernel(x, edge_index, edge_attr, w1a, b1a, w1b, b1b, elin1_w, elin1_b, eps1,
           w2a, b2a, w2b, b2b, elin2_w, elin2_b, eps2, lin_w, lin_b):
    n, d = x.shape
    e = edge_index.shape[1]
    ch = -(-e // (NW * K))
    ch += (-ch) % RING
    e_pad = NW * ch * K

    src = edge_index[0]
    dst = edge_index[1]
    a = edge_attr[:, 0]
    pad = e_pad - e
    srcs = jnp.pad(src, (0, pad)).reshape(NW, ch, K)
    dsts = jnp.pad(dst, (0, pad), constant_values=n).reshape(NW, ch, K)
    a_fix = jnp.round(jnp.clip(a, -127.0, 127.0) * ASCALE).astype(jnp.int32)
    attrs = jnp.pad(a_fix, (0, pad)).reshape(NW, ch, K)
    sa = jnp.stack([srcs, attrs], axis=2)

    n_acc = NS * 8 * (-(-(n + 1) // (NS * 8)))

    vb1 = jnp.concatenate([elin1_w, elin1_b[None, :]], axis=0)
    agg1 = _sc_message_pass(x, sa, dsts, vb1, n, n_acc, ch)
    s1 = jnp.reshape(1.0 + eps1, (1, 1))
    h1 = _tc_mlp(x, agg1, s1, w1a, b1a, w1b, b1b)

    vb2 = jnp.concatenate([elin2_w, elin2_b[None, :]], axis=0)
    agg2 = _sc_message_pass(h1, sa, dsts, vb2, n, n_acc, ch)
    s2 = jnp.reshape(1.0 + eps2, (1, 1))
    return _tc_final(h1, agg2, s2, w2a, b2a, w2b, b2b, lin_w, lin_b)
